# Initial kernel scaffold; baseline (speedup 1.0000x reference)
import functools

import jax
import jax.numpy as jnp
from jax import lax
from jax.experimental import pallas as pl
from jax.experimental.pallas import tpu as pltpu
from jax.experimental.pallas import tpu_sc as plsc

N = 10000
E = 320000
D_IN = 128
D = 32

NC = 2
NS = 16
NW = NC * NS

CH = 128
NCHT = E // CH
NCH_BASE = NCHT // NW
NCH_EXTRA = NCHT - NCH_BASE * NW

_mesh = plsc.VectorSubcoreMesh(core_axis_name="c", subcore_axis_name="s")


@functools.partial(
    pl.kernel,
    out_type=jax.ShapeDtypeStruct((NC, N), jnp.float32),
    mesh=_mesh,
    scratch_types=[
        pltpu.VMEM((CH,), jnp.int32),
        pltpu.VMEM((CH,), jnp.float32),
        pltpu.VMEM_SHARED((N,), jnp.float32),
    ],
)
def _deg_kernel(dst_hbm, zeros_hbm, out_hbm, didx, ones, acc):
    c = lax.axis_index("c")
    s = lax.axis_index("s")
    wid = c * NS + s

    @pl.when(s == 0)
    def _():
        pltpu.sync_copy(zeros_hbm, acc)

    for i in range(CH // 16):
        ones[pl.ds(i * 16, 16)] = jnp.full((16,), 1.0, jnp.float32)
    plsc.subcore_barrier()

    nch = NCH_BASE + jnp.where(wid < NCH_EXTRA, 1, 0)

    def body(j, carry):
        off = (wid + NW * j) * CH
        pltpu.sync_copy(dst_hbm.at[pl.ds(off, CH)], didx)
        pltpu.sync_copy(ones, acc.at[didx], add=True)
        return carry

    lax.fori_loop(0, nch, body, 0)
    plsc.subcore_barrier()

    @pl.when(s == 0)
    def _():
        pltpu.sync_copy(acc, out_hbm.at[c])


@functools.partial(
    pl.kernel,
    out_type=jax.ShapeDtypeStruct((NC, N, D), jnp.float32),
    mesh=_mesh,
    scratch_types=[
        pltpu.VMEM((CH,), jnp.int32),
        pltpu.VMEM((CH,), jnp.int32),
        pltpu.VMEM((CH, D), jnp.float32),
        pltpu.VMEM_SHARED((N, D), jnp.float32),
    ],
)
def _agg_kernel(hs_hbm, src_hbm, dst_hbm, zeros_hbm, out_hbm, sidx, didx, rows, acc):
    c = lax.axis_index("c")
    s = lax.axis_index("s")
    wid = c * NS + s

    @pl.when(s == 0)
    def _():
        pltpu.sync_copy(zeros_hbm, acc)
    plsc.subcore_barrier()

    nch = NCH_BASE + jnp.where(wid < NCH_EXTRA, 1, 0)

    def body(j, carry):
        off = (wid + NW * j) * CH
        pltpu.sync_copy(src_hbm.at[pl.ds(off, CH)], sidx)
        pltpu.sync_copy(dst_hbm.at[pl.ds(off, CH)], didx)
        pltpu.sync_copy(hs_hbm.at[sidx], rows)
        pltpu.sync_copy(rows, acc.at[didx], add=True)
        return carry

    lax.fori_loop(0, nch, body, 0)
    plsc.subcore_barrier()

    @pl.when(s == 0)
    def _():
        pltpu.sync_copy(acc, out_hbm.at[c])


def _tc1_body(degp_ref, x_ref, w1_ref, hs_ref, dinv_ref):
    deg = degp_ref[0] + degp_ref[1] + 1.0
    di = lax.rsqrt(deg)
    dinv_ref[...] = di
    p = jnp.dot(x_ref[...], w1_ref[...], preferred_element_type=jnp.float32)
    hs_ref[...] = di * p


def _tc2_body(accp_ref, hs1_ref, dinv_ref, b1_ref, w2_ref, hs2_ref):
    agg = accp_ref[0] + accp_ref[1] + hs1_ref[...]
    h1 = jnp.maximum(dinv_ref[...] * agg + b1_ref[...], 0.0)
    hs2_ref[...] = dinv_ref[...] * jnp.dot(
        h1, w2_ref[...], preferred_element_type=jnp.float32)


def _tc3_body(accp_ref, hs2_ref, dinv_ref, b2_ref, out_ref):
    agg = accp_ref[0] + accp_ref[1] + hs2_ref[...]
    out_ref[...] = dinv_ref[...] * agg + b2_ref[...]


_tc1 = pl.pallas_call(
    _tc1_body,
    out_shape=(
        jax.ShapeDtypeStruct((N, D), jnp.float32),
        jax.ShapeDtypeStruct((N, 1), jnp.float32),
    ),
)

_tc2 = pl.pallas_call(
    _tc2_body,
    out_shape=jax.ShapeDtypeStruct((N, D), jnp.float32),
)

_tc3 = pl.pallas_call(
    _tc3_body,
    out_shape=jax.ShapeDtypeStruct((N, D), jnp.float32),
)


def kernel(x, edge_index, W1, b1, W2, b2):
    src = edge_index[0]
    dst = edge_index[1]
    zeros_n = jnp.zeros((N,), jnp.float32)
    zeros_nd = jnp.zeros((N, D), jnp.float32)

    degp = _deg_kernel(dst, zeros_n)
    hs1, dinv = _tc1(degp.reshape(NC, N, 1), x, W1)
    accp1 = _agg_kernel(hs1, src, dst, zeros_nd)
    hs2 = _tc2(accp1, hs1, dinv, b1.reshape(1, D), W2)
    accp2 = _agg_kernel(hs2, src, dst, zeros_nd)
    out = _tc3(accp2, hs2, dinv, b2.reshape(1, D))
    return out


# speedup vs baseline: 20.5270x; 20.5270x over previous
import functools

import jax
import jax.numpy as jnp
from jax import lax
from jax.experimental import pallas as pl
from jax.experimental.pallas import tpu as pltpu
from jax.experimental.pallas import tpu_sc as plsc

N = 10000
E = 320000
D_IN = 128
D = 32

NC = 2
NS = 16
NW = NC * NS

CH = 128
NCHT = E // CH
NCH_BASE = NCHT // NW
NCH_EXTRA = NCHT - NCH_BASE * NW

_mesh = plsc.VectorSubcoreMesh(core_axis_name="c", subcore_axis_name="s")


@functools.partial(
    pl.kernel,
    out_type=jax.ShapeDtypeStruct((NC, N), jnp.float32),
    mesh=_mesh,
    scratch_types=[
        pltpu.VMEM((CH,), jnp.int32),
        pltpu.VMEM((CH,), jnp.float32),
        pltpu.VMEM_SHARED((N,), jnp.float32),
    ],
    compiler_params=pltpu.CompilerParams(use_tc_tiling_on_sc=False),
)
def _deg_kernel(dst_hbm, zeros_hbm, out_hbm, didx, ones, acc):
    c = lax.axis_index("c")
    s = lax.axis_index("s")
    wid = c * NS + s

    @pl.when(s == 0)
    def _():
        pltpu.sync_copy(zeros_hbm, acc)

    for i in range(CH // 16):
        ones[pl.ds(i * 16, 16)] = jnp.full((16,), 1.0, jnp.float32)
    plsc.subcore_barrier()

    nch = NCH_BASE + jnp.where(wid < NCH_EXTRA, 1, 0)

    def body(j, carry):
        off = (wid + NW * j) * CH
        pltpu.sync_copy(dst_hbm.at[pl.ds(off, CH)], didx)
        pltpu.sync_copy(ones, acc.at[didx], add=True)
        return carry

    lax.fori_loop(0, nch, body, 0)
    plsc.subcore_barrier()

    @pl.when(s == 0)
    def _():
        pltpu.sync_copy(acc, out_hbm.at[c])


@functools.partial(
    pl.kernel,
    out_type=jax.ShapeDtypeStruct((NC, N, D), jnp.float32),
    mesh=_mesh,
    scratch_types=[
        pltpu.VMEM((CH,), jnp.int32),
        pltpu.VMEM((CH,), jnp.int32),
        pltpu.VMEM((CH, D), jnp.float32),
        pltpu.VMEM_SHARED((N, D), jnp.float32),
    ],
    compiler_params=pltpu.CompilerParams(use_tc_tiling_on_sc=False),
)
def _agg_kernel(hs_hbm, src_hbm, dst_hbm, zeros_hbm, out_hbm, sidx, didx, rows, acc):
    c = lax.axis_index("c")
    s = lax.axis_index("s")
    wid = c * NS + s

    @pl.when(s == 0)
    def _():
        pltpu.sync_copy(zeros_hbm, acc)
    plsc.subcore_barrier()

    nch = NCH_BASE + jnp.where(wid < NCH_EXTRA, 1, 0)

    def body(j, carry):
        off = (wid + NW * j) * CH
        pltpu.sync_copy(src_hbm.at[pl.ds(off, CH)], sidx)
        pltpu.sync_copy(dst_hbm.at[pl.ds(off, CH)], didx)
        pltpu.sync_copy(hs_hbm.at[sidx], rows)
        pltpu.sync_copy(rows, acc.at[didx], add=True)
        return carry

    lax.fori_loop(0, nch, body, 0)
    plsc.subcore_barrier()

    @pl.when(s == 0)
    def _():
        pltpu.sync_copy(acc, out_hbm.at[c])


def _tc1_body(degp_ref, x_ref, w1_ref, hs_ref, dinv_ref):
    deg = degp_ref[0] + degp_ref[1] + 1.0
    di = lax.rsqrt(deg)
    dinv_ref[...] = di
    p = jnp.dot(x_ref[...], w1_ref[...], preferred_element_type=jnp.float32)
    hs_ref[...] = di * p


def _tc2_body(accp_ref, hs1_ref, dinv_ref, b1_ref, w2_ref, hs2_ref):
    agg = accp_ref[0] + accp_ref[1] + hs1_ref[...]
    h1 = jnp.maximum(dinv_ref[...] * agg + b1_ref[...], 0.0)
    hs2_ref[...] = dinv_ref[...] * jnp.dot(
        h1, w2_ref[...], preferred_element_type=jnp.float32)


def _tc3_body(accp_ref, hs2_ref, dinv_ref, b2_ref, out_ref):
    agg = accp_ref[0] + accp_ref[1] + hs2_ref[...]
    out_ref[...] = dinv_ref[...] * agg + b2_ref[...]


_tc1 = pl.pallas_call(
    _tc1_body,
    out_shape=(
        jax.ShapeDtypeStruct((N, D), jnp.float32),
        jax.ShapeDtypeStruct((N, 1), jnp.float32),
    ),
)

_tc2 = pl.pallas_call(
    _tc2_body,
    out_shape=jax.ShapeDtypeStruct((N, D), jnp.float32),
)

_tc3 = pl.pallas_call(
    _tc3_body,
    out_shape=jax.ShapeDtypeStruct((N, D), jnp.float32),
)


def kernel(x, edge_index, W1, b1, W2, b2):
    src = edge_index[0]
    dst = edge_index[1]
    zeros_n = jnp.zeros((N,), jnp.float32)
    zeros_nd = jnp.zeros((N, D), jnp.float32)

    degp = _deg_kernel(dst, zeros_n)
    hs1, dinv = _tc1(degp.reshape(NC, N, 1), x, W1)
    accp1 = _agg_kernel(hs1, src, dst, zeros_nd)
    hs2 = _tc2(accp1, hs1, dinv, b1.reshape(1, D), W2)
    accp2 = _agg_kernel(hs2, src, dst, zeros_nd)
    out = _tc3(accp2, hs2, dinv, b2.reshape(1, D))
    return out


# speedup vs baseline: 23.8290x; 1.1609x over previous
import functools

import jax
import jax.numpy as jnp
from jax import lax
from jax.experimental import pallas as pl
from jax.experimental.pallas import tpu as pltpu
from jax.experimental.pallas import tpu_sc as plsc

N = 10000
E = 320000
D_IN = 128
D = 32

NC = 2
NS = 16
NW = NC * NS

CH = 128
R = 8
NB = 10
EPW = NB * R * CH
E_PAD = NW * EPW
N_PAD = N + 16

_mesh = plsc.VectorSubcoreMesh(core_axis_name="c", subcore_axis_name="s")
_sc_params = pltpu.CompilerParams(use_tc_tiling_on_sc=False)


@functools.partial(
    pl.kernel,
    out_type=jax.ShapeDtypeStruct((NC, N), jnp.float32),
    mesh=_mesh,
    scratch_types=[
        pltpu.VMEM((R, CH), jnp.int32),
        pltpu.VMEM((CH,), jnp.float32),
        pltpu.VMEM_SHARED((N_PAD,), jnp.float32),
        pltpu.SemaphoreType.DMA,
        pltpu.SemaphoreType.DMA,
    ],
    compiler_params=_sc_params,
)
def _deg_kernel(dst_hbm, zeros_hbm, out_hbm, didx, ones, acc, sem_i, sem_s):
    c = lax.axis_index("c")
    s = lax.axis_index("s")
    wid = c * NS + s
    wbase = wid * EPW

    @pl.when(s == 0)
    def _():
        pltpu.sync_copy(zeros_hbm, acc)

    for i in range(CH // 16):
        ones[pl.ds(i * 16, 16)] = jnp.full((16,), 1.0, jnp.float32)
    plsc.subcore_barrier()

    def body(blk, carry):
        base = wbase + blk * (R * CH)
        ic = [
            pltpu.async_copy(dst_hbm.at[pl.ds(base + r * CH, CH)],
                             didx.at[r], sem_i)
            for r in range(R)
        ]
        for cp in ic:
            cp.wait()
        sc = [
            pltpu.async_copy(ones, acc.at[didx.at[r]], sem_s, add=True)
            for r in range(R)
        ]
        for cp in sc:
            cp.wait()
        return carry

    lax.fori_loop(0, NB, body, 0)
    plsc.subcore_barrier()

    @pl.when(s == 0)
    def _():
        pltpu.sync_copy(acc.at[pl.ds(0, N)], out_hbm.at[c])


@functools.partial(
    pl.kernel,
    out_type=jax.ShapeDtypeStruct((NC, N, D), jnp.float32),
    mesh=_mesh,
    scratch_types=[
        pltpu.VMEM((R, CH), jnp.int32),
        pltpu.VMEM((R, CH), jnp.int32),
        pltpu.VMEM((R, CH, D), jnp.float32),
        pltpu.VMEM_SHARED((N_PAD, D), jnp.float32),
        pltpu.SemaphoreType.DMA,
        pltpu.SemaphoreType.DMA,
        pltpu.SemaphoreType.DMA,
    ],
    compiler_params=_sc_params,
)
def _agg_kernel(hs_hbm, src_hbm, dst_hbm, zeros_hbm, out_hbm,
                sidx, didx, rows, acc, sem_i, sem_g, sem_s):
    c = lax.axis_index("c")
    s = lax.axis_index("s")
    wid = c * NS + s
    wbase = wid * EPW

    @pl.when(s == 0)
    def _():
        pltpu.sync_copy(zeros_hbm, acc)
    plsc.subcore_barrier()

    def body(blk, carry):
        base = wbase + blk * (R * CH)
        ic = []
        for r in range(R):
            off = base + r * CH
            ic.append(pltpu.async_copy(src_hbm.at[pl.ds(off, CH)],
                                       sidx.at[r], sem_i))
            ic.append(pltpu.async_copy(dst_hbm.at[pl.ds(off, CH)],
                                       didx.at[r], sem_i))
        for cp in ic:
            cp.wait()
        gc = [
            pltpu.async_copy(hs_hbm.at[sidx.at[r]], rows.at[r], sem_g)
            for r in range(R)
        ]
        for cp in gc:
            cp.wait()
        sc = [
            pltpu.async_copy(rows.at[r], acc.at[didx.at[r]], sem_s, add=True)
            for r in range(R)
        ]
        for cp in sc:
            cp.wait()
        return carry

    lax.fori_loop(0, NB, body, 0)
    plsc.subcore_barrier()

    @pl.when(s == 0)
    def _():
        pltpu.sync_copy(acc.at[pl.ds(0, N)], out_hbm.at[c])


def _tc1_body(degp_ref, x_ref, w1_ref, hs_ref, dinv_ref):
    deg = degp_ref[0] + degp_ref[1] + 1.0
    di = lax.rsqrt(deg)
    dinv_ref[...] = di
    p = jnp.dot(x_ref[...], w1_ref[...], preferred_element_type=jnp.float32)
    hs_ref[...] = di * p


def _tc2_body(accp_ref, hs1_ref, dinv_ref, b1_ref, w2_ref, hs2_ref):
    agg = accp_ref[0] + accp_ref[1] + hs1_ref[...]
    h1 = jnp.maximum(dinv_ref[...] * agg + b1_ref[...], 0.0)
    hs2_ref[...] = dinv_ref[...] * jnp.dot(
        h1, w2_ref[...], preferred_element_type=jnp.float32)


def _tc3_body(accp_ref, hs2_ref, dinv_ref, b2_ref, out_ref):
    agg = accp_ref[0] + accp_ref[1] + hs2_ref[...]
    out_ref[...] = dinv_ref[...] * agg + b2_ref[...]


_tc1 = pl.pallas_call(
    _tc1_body,
    out_shape=(
        jax.ShapeDtypeStruct((N, D), jnp.float32),
        jax.ShapeDtypeStruct((N, 1), jnp.float32),
    ),
)

_tc2 = pl.pallas_call(
    _tc2_body,
    out_shape=jax.ShapeDtypeStruct((N, D), jnp.float32),
)

_tc3 = pl.pallas_call(
    _tc3_body,
    out_shape=jax.ShapeDtypeStruct((N, D), jnp.float32),
)


def kernel(x, edge_index, W1, b1, W2, b2):
    pad = E_PAD - E
    src = jnp.concatenate([edge_index[0], jnp.zeros((pad,), jnp.int32)])
    dst = jnp.concatenate([edge_index[1], jnp.full((pad,), N, jnp.int32)])
    zeros_n = jnp.zeros((N_PAD,), jnp.float32)
    zeros_nd = jnp.zeros((N_PAD, D), jnp.float32)

    degp = _deg_kernel(dst, zeros_n)
    hs1, dinv = _tc1(degp.reshape(NC, N, 1), x, W1)
    accp1 = _agg_kernel(hs1, src, dst, zeros_nd)
    hs2 = _tc2(accp1, hs1, dinv, b1.reshape(1, D), W2)
    accp2 = _agg_kernel(hs2, src, dst, zeros_nd)
    out = _tc3(accp2, hs2, dinv, b2.reshape(1, D))
    return out


# speedup vs baseline: 24.7811x; 1.0400x over previous
import functools

import jax
import jax.numpy as jnp
from jax import lax
from jax.experimental import pallas as pl
from jax.experimental.pallas import tpu as pltpu
from jax.experimental.pallas import tpu_sc as plsc

N = 10000
E = 320000
D_IN = 128
D = 32

NC = 2
NS = 16
NW = NC * NS

CH = 128
R = 8
NB = 10
EPW = NB * R * CH
E_PAD = NW * EPW
N_DUMP = 2048
N_PAD = N + N_DUMP

_mesh = plsc.VectorSubcoreMesh(core_axis_name="c", subcore_axis_name="s")
_sc_params = pltpu.CompilerParams(use_tc_tiling_on_sc=False)


@functools.partial(
    pl.kernel,
    out_type=jax.ShapeDtypeStruct((NC, N), jnp.float32),
    mesh=_mesh,
    scratch_types=[
        pltpu.VMEM((R, CH), jnp.int32),
        pltpu.VMEM((CH,), jnp.float32),
        pltpu.VMEM_SHARED((N_PAD,), jnp.float32),
        pltpu.SemaphoreType.DMA,
        pltpu.SemaphoreType.DMA,
    ],
    compiler_params=_sc_params,
)
def _deg_kernel(dst_hbm, zeros_hbm, out_hbm, didx, ones, acc, sem_i, sem_s):
    c = lax.axis_index("c")
    s = lax.axis_index("s")
    wid = c * NS + s
    wbase = wid * EPW

    @pl.when(s == 0)
    def _():
        pltpu.sync_copy(zeros_hbm, acc)

    for i in range(CH // 16):
        ones[pl.ds(i * 16, 16)] = jnp.full((16,), 1.0, jnp.float32)
    plsc.subcore_barrier()

    def body(blk, carry):
        base = wbase + blk * (R * CH)
        ic = [
            pltpu.async_copy(dst_hbm.at[pl.ds(base + r * CH, CH)],
                             didx.at[r], sem_i)
            for r in range(R)
        ]
        for cp in ic:
            cp.wait()
        sc = [
            pltpu.async_copy(ones, acc.at[didx.at[r]], sem_s, add=True)
            for r in range(R)
        ]
        for cp in sc:
            cp.wait()
        return carry

    lax.fori_loop(0, NB, body, 0)
    plsc.subcore_barrier()

    @pl.when(s == 0)
    def _():
        pltpu.sync_copy(acc.at[pl.ds(0, N)], out_hbm.at[c])


@functools.partial(
    pl.kernel,
    out_type=jax.ShapeDtypeStruct((NC, N, D), jnp.float32),
    mesh=_mesh,
    scratch_types=[
        pltpu.VMEM((R, CH), jnp.int32),
        pltpu.VMEM((R, CH), jnp.int32),
        pltpu.VMEM((R, CH, D), jnp.float32),
        pltpu.VMEM_SHARED((N_PAD, D), jnp.float32),
        pltpu.SemaphoreType.DMA,
        pltpu.SemaphoreType.DMA,
        pltpu.SemaphoreType.DMA,
    ],
    compiler_params=_sc_params,
)
def _agg_kernel(hs_hbm, src_hbm, dst_hbm, zeros_hbm, out_hbm,
                sidx, didx, rows, acc, sem_i, sem_g, sem_s):
    c = lax.axis_index("c")
    s = lax.axis_index("s")
    wid = c * NS + s
    wbase = wid * EPW

    @pl.when(s == 0)
    def _():
        pltpu.sync_copy(zeros_hbm, acc)
    plsc.subcore_barrier()

    def body(blk, carry):
        base = wbase + blk * (R * CH)
        ic = []
        for r in range(R):
            off = base + r * CH
            ic.append(pltpu.async_copy(src_hbm.at[pl.ds(off, CH)],
                                       sidx.at[r], sem_i))
            ic.append(pltpu.async_copy(dst_hbm.at[pl.ds(off, CH)],
                                       didx.at[r], sem_i))
        for cp in ic:
            cp.wait()
        gc = [
            pltpu.async_copy(hs_hbm.at[sidx.at[r]], rows.at[r], sem_g)
            for r in range(R)
        ]
        for cp in gc:
            cp.wait()
        sc = [
            pltpu.async_copy(rows.at[r], acc.at[didx.at[r]], sem_s, add=True)
            for r in range(R)
        ]
        for cp in sc:
            cp.wait()
        return carry

    lax.fori_loop(0, NB, body, 0)
    plsc.subcore_barrier()

    @pl.when(s == 0)
    def _():
        pltpu.sync_copy(acc.at[pl.ds(0, N)], out_hbm.at[c])


def _tc1_body(degp_ref, x_ref, w1_ref, hs_ref, dinv_ref):
    deg = degp_ref[0] + degp_ref[1] + 1.0
    di = lax.rsqrt(deg)
    dinv_ref[...] = di
    p = jnp.dot(x_ref[...], w1_ref[...], preferred_element_type=jnp.float32)
    hs_ref[...] = di * p


def _tc2_body(accp_ref, hs1_ref, dinv_ref, b1_ref, w2_ref, hs2_ref):
    agg = accp_ref[0] + accp_ref[1] + hs1_ref[...]
    h1 = jnp.maximum(dinv_ref[...] * agg + b1_ref[...], 0.0)
    hs2_ref[...] = dinv_ref[...] * jnp.dot(
        h1, w2_ref[...], preferred_element_type=jnp.float32)


def _tc3_body(accp_ref, hs2_ref, dinv_ref, b2_ref, out_ref):
    agg = accp_ref[0] + accp_ref[1] + hs2_ref[...]
    out_ref[...] = dinv_ref[...] * agg + b2_ref[...]


_tc1 = pl.pallas_call(
    _tc1_body,
    out_shape=(
        jax.ShapeDtypeStruct((N, D), jnp.float32),
        jax.ShapeDtypeStruct((N, 1), jnp.float32),
    ),
)

_tc2 = pl.pallas_call(
    _tc2_body,
    out_shape=jax.ShapeDtypeStruct((N, D), jnp.float32),
)

_tc3 = pl.pallas_call(
    _tc3_body,
    out_shape=jax.ShapeDtypeStruct((N, D), jnp.float32),
)


def kernel(x, edge_index, W1, b1, W2, b2):
    pad = E_PAD - E
    src = jnp.concatenate([edge_index[0], jnp.zeros((pad,), jnp.int32)])
    dst = jnp.concatenate(
        [edge_index[1], N + (jnp.arange(pad, dtype=jnp.int32) % N_DUMP)])
    zeros_n = jnp.zeros((N_PAD,), jnp.float32)
    zeros_nd = jnp.zeros((N_PAD, D), jnp.float32)

    degp = _deg_kernel(dst, zeros_n)
    hs1, dinv = _tc1(degp.reshape(NC, N, 1), x, W1)
    accp1 = _agg_kernel(hs1, src, dst, zeros_nd)
    hs2 = _tc2(accp1, hs1, dinv, b1.reshape(1, D), W2)
    accp2 = _agg_kernel(hs2, src, dst, zeros_nd)
    out = _tc3(accp2, hs2, dinv, b2.reshape(1, D))
    return out


# speedup vs baseline: 43.9217x; 1.7724x over previous
import functools

import jax
import jax.numpy as jnp
from jax import lax
from jax.experimental import pallas as pl
from jax.experimental.pallas import tpu as pltpu
from jax.experimental.pallas import tpu_sc as plsc

N = 10000
E = 320000
D_IN = 128
D = 32

NC = 2
NS = 16
NW = NC * NS

CH = 128
R = 8
NB = 10
EPW = NB * R * CH
E_PAD = NW * EPW
N_DUMP = 2048
N_PAD = N + N_DUMP

_mesh = plsc.VectorSubcoreMesh(core_axis_name="c", subcore_axis_name="s")
_sc_params = pltpu.CompilerParams(use_tc_tiling_on_sc=False)


@functools.partial(
    pl.kernel,
    out_type=jax.ShapeDtypeStruct((NC, N), jnp.float32),
    mesh=_mesh,
    scratch_types=[
        pltpu.VMEM((R, CH), jnp.int32),
        pltpu.VMEM((CH,), jnp.float32),
        pltpu.VMEM_SHARED((N_PAD,), jnp.float32),
        pltpu.SemaphoreType.DMA,
        pltpu.SemaphoreType.DMA,
    ],
    compiler_params=_sc_params,
)
def _deg_kernel(dst_hbm, zeros_hbm, out_hbm, didx, ones, acc, sem_i, sem_s):
    c = lax.axis_index("c")
    s = lax.axis_index("s")
    wid = c * NS + s
    wbase = wid * EPW

    @pl.when(s == 0)
    def _():
        pltpu.sync_copy(zeros_hbm, acc)

    for i in range(CH // 16):
        ones[pl.ds(i * 16, 16)] = jnp.full((16,), 1.0, jnp.float32)
    plsc.subcore_barrier()

    def body(blk, carry):
        base = wbase + blk * (R * CH)
        ic = [
            pltpu.async_copy(dst_hbm.at[pl.ds(base + r * CH, CH)],
                             didx.at[r], sem_i)
            for r in range(R)
        ]
        for cp in ic:
            cp.wait()
        sc = [
            pltpu.async_copy(ones, acc.at[didx.at[r]], sem_s, add=True)
            for r in range(R)
        ]
        for cp in sc:
            cp.wait()
        return carry

    lax.fori_loop(0, NB, body, 0)
    plsc.subcore_barrier()

    @pl.when(s == 0)
    def _():
        pltpu.sync_copy(acc.at[pl.ds(0, N)], out_hbm.at[c])


@functools.partial(
    pl.kernel,
    out_type=jax.ShapeDtypeStruct((NC, N, D), jnp.float32),
    mesh=_mesh,
    scratch_types=[
        pltpu.VMEM((R, CH), jnp.int32),
        pltpu.VMEM((R, CH), jnp.int32),
        pltpu.VMEM((R, CH, D), jnp.float32),
        pltpu.VMEM_SHARED((N_PAD, D), jnp.float32),
        pltpu.VMEM_SHARED((N, D), jnp.float32),
        pltpu.SemaphoreType.DMA,
        pltpu.SemaphoreType.DMA,
        pltpu.SemaphoreType.DMA,
    ],
    compiler_params=_sc_params,
)
def _agg_kernel(hs_hbm, src_hbm, dst_hbm, zeros_hbm, out_hbm,
                sidx, didx, rows, acc, table, sem_i, sem_g, sem_s):
    c = lax.axis_index("c")
    s = lax.axis_index("s")
    wid = c * NS + s
    wbase = wid * EPW
    rows_per_tile = N // NS

    @pl.when(s == 0)
    def _():
        pltpu.sync_copy(zeros_hbm, acc)
    pltpu.sync_copy(hs_hbm.at[pl.ds(s * rows_per_tile, rows_per_tile)],
                    table.at[pl.ds(s * rows_per_tile, rows_per_tile)])
    plsc.subcore_barrier()

    def body(blk, carry):
        base = wbase + blk * (R * CH)
        ic = []
        for r in range(R):
            off = base + r * CH
            ic.append(pltpu.async_copy(src_hbm.at[pl.ds(off, CH)],
                                       sidx.at[r], sem_i))
            ic.append(pltpu.async_copy(dst_hbm.at[pl.ds(off, CH)],
                                       didx.at[r], sem_i))
        for cp in ic:
            cp.wait()
        gc = [
            pltpu.async_copy(table.at[sidx.at[r]], rows.at[r], sem_g)
            for r in range(R)
        ]
        for cp in gc:
            cp.wait()
        sc = [
            pltpu.async_copy(rows.at[r], acc.at[didx.at[r]], sem_s, add=True)
            for r in range(R)
        ]
        for cp in sc:
            cp.wait()
        return carry

    lax.fori_loop(0, NB, body, 0)
    plsc.subcore_barrier()

    @pl.when(s == 0)
    def _():
        pltpu.sync_copy(acc.at[pl.ds(0, N)], out_hbm.at[c])


def _tc1_body(degp_ref, x_ref, w1_ref, hs_ref, dinv_ref):
    deg = degp_ref[0] + degp_ref[1] + 1.0
    di = lax.rsqrt(deg)
    dinv_ref[...] = di
    p = jnp.dot(x_ref[...], w1_ref[...], preferred_element_type=jnp.float32)
    hs_ref[...] = di * p


def _tc2_body(accp_ref, hs1_ref, dinv_ref, b1_ref, w2_ref, hs2_ref):
    agg = accp_ref[0] + accp_ref[1] + hs1_ref[...]
    h1 = jnp.maximum(dinv_ref[...] * agg + b1_ref[...], 0.0)
    hs2_ref[...] = dinv_ref[...] * jnp.dot(
        h1, w2_ref[...], preferred_element_type=jnp.float32)


def _tc3_body(accp_ref, hs2_ref, dinv_ref, b2_ref, out_ref):
    agg = accp_ref[0] + accp_ref[1] + hs2_ref[...]
    out_ref[...] = dinv_ref[...] * agg + b2_ref[...]


_tc1 = pl.pallas_call(
    _tc1_body,
    out_shape=(
        jax.ShapeDtypeStruct((N, D), jnp.float32),
        jax.ShapeDtypeStruct((N, 1), jnp.float32),
    ),
)

_tc2 = pl.pallas_call(
    _tc2_body,
    out_shape=jax.ShapeDtypeStruct((N, D), jnp.float32),
)

_tc3 = pl.pallas_call(
    _tc3_body,
    out_shape=jax.ShapeDtypeStruct((N, D), jnp.float32),
)


def kernel(x, edge_index, W1, b1, W2, b2):
    pad = E_PAD - E
    src = jnp.concatenate([edge_index[0], jnp.zeros((pad,), jnp.int32)])
    dst = jnp.concatenate(
        [edge_index[1], N + (jnp.arange(pad, dtype=jnp.int32) % N_DUMP)])
    zeros_n = jnp.zeros((N_PAD,), jnp.float32)
    zeros_nd = jnp.zeros((N_PAD, D), jnp.float32)

    degp = _deg_kernel(dst, zeros_n)
    hs1, dinv = _tc1(degp.reshape(NC, N, 1), x, W1)
    accp1 = _agg_kernel(hs1, src, dst, zeros_nd)
    hs2 = _tc2(accp1, hs1, dinv, b1.reshape(1, D), W2)
    accp2 = _agg_kernel(hs2, src, dst, zeros_nd)
    out = _tc3(accp2, hs2, dinv, b2.reshape(1, D))
    return out


# speedup vs baseline: 51.3060x; 1.1681x over previous
import jax
import jax.numpy as jnp
from jax import lax
from jax.experimental import pallas as pl
from jax.experimental.pallas import tpu as pltpu
from jax.experimental.pallas import tpu_sc as plsc

N = 10000
E = 320000
D_IN = 128
D = 32

NC = 2
NS = 16
NW = NC * NS

CH = 128
NCHT = E // CH
R = 6
NB = 13
NCH_BASE = NB * R
NX = NCHT - NCH_BASE * NW
RPT = N // NS

_mesh = plsc.VectorSubcoreMesh(core_axis_name="c", subcore_axis_name="s")
_sc_params = pltpu.CompilerParams(use_tc_tiling_on_sc=False)


def _worker_base(wid):
    return (wid * NCH_BASE + jnp.minimum(wid, NX)) * CH


def _deg_body(dst_hbm, zeros_hbm, out_hbm, didx, ones, acc, sem_i, sem_s):
    c = lax.axis_index("c")
    s = lax.axis_index("s")
    wid = c * NS + s
    wbase = _worker_base(wid)

    @pl.when(s == 0)
    def _():
        pltpu.sync_copy(zeros_hbm, acc)

    for i in range(CH // 16):
        ones[pl.ds(i * 16, 16)] = jnp.full((16,), 1.0, jnp.float32)
    plsc.subcore_barrier()

    def body(blk, carry):
        base = wbase + blk * (R * CH)
        ic = [
            pltpu.async_copy(dst_hbm.at[pl.ds(base + r * CH, CH)],
                             didx.at[r], sem_i)
            for r in range(R)
        ]
        for cp in ic:
            cp.wait()
        sc = [
            pltpu.async_copy(ones, acc.at[didx.at[r]], sem_s, add=True)
            for r in range(R)
        ]
        for cp in sc:
            cp.wait()
        return carry

    lax.fori_loop(0, NB, body, 0)

    @pl.when(wid < NX)
    def _():
        off = (NCH_BASE * NW + wid) * CH
        pltpu.sync_copy(dst_hbm.at[pl.ds(off, CH)], didx.at[0])
        pltpu.sync_copy(ones, acc.at[didx.at[0]], add=True)

    plsc.subcore_barrier()

    @pl.when(s == 0)
    def _():
        pltpu.sync_copy(acc, out_hbm.at[c])


_deg_kernel = pl.kernel(
    _deg_body,
    out_type=jax.ShapeDtypeStruct((NC, N), jnp.float32),
    mesh=_mesh,
    scratch_types=[
        pltpu.VMEM((R, CH), jnp.int32),
        pltpu.VMEM((CH,), jnp.float32),
        pltpu.VMEM_SHARED((N,), jnp.float32),
        pltpu.SemaphoreType.DMA,
        pltpu.SemaphoreType.DMA,
    ],
    compiler_params=_sc_params,
)


def _agg_body(hs_hbm, src_hbm, dst_hbm, zeros_hbm, out_hbm,
              sidx, didx, rows, acc, table,
              sem_i0, sem_i1, sem_g0, sem_g1, sem_s0, sem_s1):
    c = lax.axis_index("c")
    s = lax.axis_index("s")
    wid = c * NS + s
    wbase = _worker_base(wid)
    sem_i = (sem_i0, sem_i1)
    sem_g = (sem_g0, sem_g1)
    sem_s = (sem_s0, sem_s1)

    pltpu.sync_copy(zeros_hbm.at[pl.ds(s * RPT, RPT)],
                    acc.at[pl.ds(s * RPT, RPT)])
    pltpu.sync_copy(hs_hbm.at[pl.ds(s * RPT, RPT)],
                    table.at[pl.ds(s * RPT, RPT)])
    plsc.subcore_barrier()

    def fire_idx(k):
        g = k % 2
        cps = []
        for r in range(R):
            off = wbase + (k * R + r) * CH
            cps.append(pltpu.async_copy(src_hbm.at[pl.ds(off, CH)],
                                        sidx.at[g, r], sem_i[g]))
            cps.append(pltpu.async_copy(dst_hbm.at[pl.ds(off, CH)],
                                        didx.at[g, r], sem_i[g]))
        return cps

    def fire_gather(k):
        g = k % 2
        return [
            pltpu.async_copy(table.at[sidx.at[g, r]], rows.at[g, r], sem_g[g])
            for r in range(R)
        ]

    def fire_scatter(k):
        g = k % 2
        return [
            pltpu.async_copy(rows.at[g, r], acc.at[didx.at[g, r]],
                             sem_s[g], add=True)
            for r in range(R)
        ]

    idx_d = {0: fire_idx(0)}
    for cp in idx_d[0]:
        cp.wait()
    gat_d = {0: fire_gather(0)}
    idx_d[1] = fire_idx(1)
    sca_d = {}
    for k in range(NB):
        for cp in gat_d[k]:
            cp.wait()
        sca_d[k] = fire_scatter(k)
        if k + 1 < NB:
            for cp in idx_d[k + 1]:
                cp.wait()
            if k >= 1:
                for cp in sca_d[k - 1]:
                    cp.wait()
            gat_d[k + 1] = fire_gather(k + 1)
            if k + 2 < NB:
                idx_d[k + 2] = fire_idx(k + 2)
    for cp in sca_d[NB - 2]:
        cp.wait()
    for cp in sca_d[NB - 1]:
        cp.wait()

    @pl.when(wid < NX)
    def _():
        off = (NCH_BASE * NW + wid) * CH
        pltpu.sync_copy(src_hbm.at[pl.ds(off, CH)], sidx.at[0, 0])
        pltpu.sync_copy(dst_hbm.at[pl.ds(off, CH)], didx.at[0, 0])
        pltpu.sync_copy(table.at[sidx.at[0, 0]], rows.at[0, 0])
        pltpu.sync_copy(rows.at[0, 0], acc.at[didx.at[0, 0]], add=True)

    plsc.subcore_barrier()

    @pl.when(s == 0)
    def _():
        pltpu.sync_copy(acc, out_hbm.at[c])


_agg_kernel = pl.kernel(
    _agg_body,
    out_type=jax.ShapeDtypeStruct((NC, N, D), jnp.float32),
    mesh=_mesh,
    scratch_types=[
        pltpu.VMEM((2, R, CH), jnp.int32),
        pltpu.VMEM((2, R, CH), jnp.int32),
        pltpu.VMEM((2, R, CH, D), jnp.float32),
        pltpu.VMEM_SHARED((N, D), jnp.float32),
        pltpu.VMEM_SHARED((N, D), jnp.float32),
        pltpu.SemaphoreType.DMA,
        pltpu.SemaphoreType.DMA,
        pltpu.SemaphoreType.DMA,
        pltpu.SemaphoreType.DMA,
        pltpu.SemaphoreType.DMA,
        pltpu.SemaphoreType.DMA,
    ],
    compiler_params=_sc_params,
)


def _tc1_body(degp_ref, x_ref, w1_ref, hs_ref, dinv_ref):
    deg = degp_ref[0:1, :] + degp_ref[1:2, :] + 1.0
    di = jnp.transpose(lax.rsqrt(deg))
    dinv_ref[...] = di
    p = jnp.dot(x_ref[...], w1_ref[...], preferred_element_type=jnp.float32)
    hs_ref[...] = di * p


def _tc2_body(accp_ref, hs1_ref, dinv_ref, b1_ref, w2_ref, hs2_ref):
    agg = accp_ref[0] + accp_ref[1] + hs1_ref[...]
    h1 = jnp.maximum(dinv_ref[...] * agg + b1_ref[...], 0.0)
    hs2_ref[...] = dinv_ref[...] * jnp.dot(
        h1, w2_ref[...], preferred_element_type=jnp.float32)


def _tc3_body(accp_ref, hs2_ref, dinv_ref, b2_ref, out_ref):
    agg = accp_ref[0] + accp_ref[1] + hs2_ref[...]
    out_ref[...] = dinv_ref[...] * agg + b2_ref[...]


_tc1 = pl.pallas_call(
    _tc1_body,
    out_shape=(
        jax.ShapeDtypeStruct((N, D), jnp.float32),
        jax.ShapeDtypeStruct((N, 1), jnp.float32),
    ),
)

_tc2 = pl.pallas_call(
    _tc2_body,
    out_shape=jax.ShapeDtypeStruct((N, D), jnp.float32),
)

_tc3 = pl.pallas_call(
    _tc3_body,
    out_shape=jax.ShapeDtypeStruct((N, D), jnp.float32),
)


def kernel(x, edge_index, W1, b1, W2, b2):
    src = edge_index[0]
    dst = edge_index[1]
    zeros_n = jnp.zeros((N,), jnp.float32)
    zeros_nd = jnp.zeros((N, D), jnp.float32)
    b1r = b1.reshape(1, D)
    b2r = b2.reshape(1, D)

    degp = _deg_kernel(dst, zeros_n)
    hs1, dinv = _tc1(degp, x, W1)
    accp1 = _agg_kernel(hs1, src, dst, zeros_nd)
    hs2 = _tc2(accp1, hs1, dinv, b1r, W2)
    accp2 = _agg_kernel(hs2, src, dst, zeros_nd)
    out = _tc3(accp2, hs2, dinv, b2r)
    return out
